# baseline (speedup 1.0000x reference)
import jax
import jax.numpy as jnp
from jax.experimental import pallas as pl
from jax.experimental.pallas import tpu as pltpu

N_EXPERTS = 64
TOPK = 8
BLK = 4096


def _router_kernel(x_ref, wcat_ref, b1_ref, b2_ref, out_ref, idx_ref):
    wb = wcat_ref[...].astype(jnp.bfloat16)
    xb = x_ref[0].astype(jnp.bfloat16)
    ll = jax.lax.dot_general(wb, xb, (((0,), (1,)), ((), ())),
                             preferred_element_type=jnp.float32)
    l1 = ll[:N_EXPERTS] + b1_ref[...]
    l2 = ll[N_EXPERTS:] + b2_ref[...]
    logits = l1 + l2
    cols = logits.shape[1]
    ids = jax.lax.broadcasted_iota(jnp.int32, (N_EXPERTS, cols), 0).astype(
        jnp.float32)
    v = logits
    idx_rows = []
    m1 = None
    for k in range(TOPK):
        m = jnp.max(v, axis=0, keepdims=True)
        if k == 0:
            m1 = m
        eq = v == m
        idx = jnp.min(jnp.where(eq, ids, float(N_EXPERTS)), axis=0,
                      keepdims=True)
        idx_rows.append(idx)
        v = jnp.where(eq, -jnp.inf, v)
    idx_out = jnp.concatenate(idx_rows, axis=0)
    p = jnp.where(v == -jnp.inf, jnp.exp(logits - m1), 0.0)
    p = p / jnp.sum(p, axis=0, keepdims=True)
    out_ref[...] = p.T[None]
    idx_ref[...] = idx_out.astype(jnp.int32).T[None]


@jax.jit
def kernel(x, W1, b1, W2, b2):
    B, S, E = x.shape
    wcat = jnp.concatenate([W1, W2], axis=1)
    b1r = b1.reshape(N_EXPERTS, 1)
    b2r = b2.reshape(N_EXPERTS, 1)
    router, idx = pl.pallas_call(
        _router_kernel,
        grid=(B, S // BLK),
        in_specs=[
            pl.BlockSpec((1, BLK, E), lambda b, i: (b, i, 0)),
            pl.BlockSpec((E, 2 * N_EXPERTS), lambda b, i: (0, 0)),
            pl.BlockSpec((N_EXPERTS, 1), lambda b, i: (0, 0)),
            pl.BlockSpec((N_EXPERTS, 1), lambda b, i: (0, 0)),
        ],
        out_specs=[
            pl.BlockSpec((1, BLK, N_EXPERTS), lambda b, i: (b, i, 0)),
            pl.BlockSpec((1, BLK, TOPK), lambda b, i: (b, i, 0)),
        ],
        out_shape=[
            jax.ShapeDtypeStruct((B, S, N_EXPERTS), jnp.float32),
            jax.ShapeDtypeStruct((B, S, TOPK), jnp.int32),
        ],
        compiler_params=pltpu.CompilerParams(
            dimension_semantics=("parallel", "parallel"),
        ),
    )(x, wcat, b1r, b2r)
    return router, idx
